# Initial kernel scaffold; baseline (speedup 1.0000x reference)
import jax
import jax.numpy as jnp
from jax.experimental import pallas as pl


def kernel(x, edge_index, batch, W1, att_src1, att_dst1, bias1, W2, att_src2, att_dst2, bias2):
    raise NotImplementedError("write your pallas kernel here")


# speedup vs baseline: 46.5680x; 46.5680x over previous
import functools

import jax
import jax.numpy as jnp
from jax import lax
from jax.experimental import pallas as pl
from jax.experimental.pallas import tpu as pltpu
from jax.experimental.pallas import tpu_sc as plsc

N = 10000
NP = 10240
E = 320000
EP = 327680
C = 80
NCHUNK = EP // 32 // C
F = 48
ONES_COL = 41

_mesh = plsc.VectorSubcoreMesh(core_axis_name="c", subcore_axis_name="s")

_GDN = lax.GatherDimensionNumbers(
    offset_dims=(), collapsed_slice_dims=(0,), start_index_map=(0,))


def _splat(vec16, idx16):
    return lax.gather(vec16, idx16[:, None], _GDN, (1,),
                      mode=lax.GatherScatterMode.PROMISE_IN_BOUNDS)


def _tca_body(x_ref, w_ref, a2_ref, h_ref, ab_ref):
    h = jnp.dot(x_ref[...], w_ref[...], preferred_element_type=jnp.float32)
    col = lax.broadcasted_iota(jnp.int32, h.shape, 1)
    hext = jnp.where(col == ONES_COL, 1.0, h)
    h_ref[...] = hext
    ab_ref[...] = jnp.dot(hext, a2_ref[...], preferred_element_type=jnp.float32)


def _tc_a(x_pad, w1t, a2mat):
    return pl.pallas_call(
        _tca_body,
        out_shape=[
            jax.ShapeDtypeStruct((NP, F), jnp.float32),
            jax.ShapeDtypeStruct((NP, 2), jnp.float32),
        ],
    )(x_pad, w1t, a2mat)


@functools.partial(
    pl.kernel,
    mesh=_mesh,
    compiler_params=pltpu.CompilerParams(needs_layout_passes=False, use_tc_tiling_on_sc=False),
    out_type=jax.ShapeDtypeStruct((2, NP, F), jnp.float32),
    scratch_types=[
        pltpu.VMEM((NP,), jnp.float32),
        pltpu.VMEM((NP,), jnp.float32),
        pltpu.VMEM((NCHUNK, C), jnp.int32),
        pltpu.VMEM((NCHUNK, C), jnp.int32),
        pltpu.VMEM((C,), jnp.float32),
        pltpu.VMEM((C, F), jnp.float32),
        pltpu.VMEM_SHARED((NP, F), jnp.float32),
        pltpu.SemaphoreType.DMA,
    ],
)
def _sc_a(as_hbm, ad_hbm, src_hbm, dst_hbm, hext_hbm, zrows_hbm, acc_out,
          as_v, ad_v, src_v, dst_v, exbuf, rows_v, acc_sh, sem):
    cid = lax.axis_index("c")
    sid = lax.axis_index("s")
    rowbase = (cid * 16 + sid) * NCHUNK

    pltpu.sync_copy(as_hbm, as_v)
    pltpu.sync_copy(ad_hbm, ad_v)
    pltpu.sync_copy(src_hbm.at[pl.ds(rowbase, NCHUNK)], src_v)
    pltpu.sync_copy(dst_hbm.at[pl.ds(rowbase, NCHUNK)], dst_v)

    for k in range(5):
        pltpu.sync_copy(zrows_hbm, acc_sh.at[pl.ds(sid * 640 + k * 128, 128)])
    plsc.subcore_barrier()

    lane_ids = [jnp.full((16,), l, jnp.int32) for l in range(16)]

    def chunk_body(ci, carry):
        for g in range(C // 16):
            sv = src_v[ci, pl.ds(g * 16, 16)]
            dv = dst_v[ci, pl.ds(g * 16, 16)]
            al = (plsc.load_gather(as_v, [sv])
                  + plsc.load_gather(ad_v, [dv]))
            al = jnp.where(al >= 0, al, al * jnp.float32(0.2))
            exbuf[pl.ds(g * 16, 16)] = jnp.exp(al)
        pltpu.async_copy(hext_hbm.at[src_v.at[ci]], rows_v, sem).wait()
        for g in range(C // 16):
            exv = exbuf[pl.ds(g * 16, 16)]
            for l in range(16):
                spl = _splat(exv, lane_ids[l])
                e = g * 16 + l
                for j in range(F // 16):
                    r = rows_v[e, pl.ds(j * 16, 16)]
                    rows_v[e, pl.ds(j * 16, 16)] = r * spl
        pltpu.sync_copy(rows_v, acc_sh.at[dst_v.at[ci]], add=True)
        return carry

    lax.fori_loop(0, NCHUNK, chunk_body, 0)
    plsc.subcore_barrier()

    pltpu.sync_copy(acc_sh.at[pl.ds(sid * 640, 640)],
                    acc_out.at[cid, pl.ds(sid * 640, 640)])


def _tcb_body(acc_ref, ab_ref, h_ref, b1_ref, w2_ref, a2v_ref,
              h2_ref, hs2_ref, hd2_ref):
    A = acc_ref[...]
    ab = ab_ref[...]
    al = ab[:, 0:1] + ab[:, 1:2]
    al = jnp.where(al >= 0, al, al * jnp.float32(0.2))
    exs = jnp.exp(al)
    accf = A[0] + A[1] + exs * h_ref[...]
    den = accf[:, ONES_COL:ONES_COL + 1]
    out1 = jnp.maximum(accf / (den + 1e-16) + b1_ref[...], 0.0)
    h2 = jnp.sum(out1 * w2_ref[...], axis=1, keepdims=True)
    h2_ref[...] = h2
    hs2_ref[...] = h2 * a2v_ref[0, 0]
    hd2_ref[...] = h2 * a2v_ref[0, 1]


def _tc_b(acc, ab, hext, b1p, w2p, a2v):
    return pl.pallas_call(
        _tcb_body,
        out_shape=[
            jax.ShapeDtypeStruct((NP, 1), jnp.float32),
            jax.ShapeDtypeStruct((NP, 1), jnp.float32),
            jax.ShapeDtypeStruct((NP, 1), jnp.float32),
        ],
    )(acc, ab, hext, b1p, w2p, a2v)


@functools.partial(
    pl.kernel,
    mesh=_mesh,
    compiler_params=pltpu.CompilerParams(needs_layout_passes=False, use_tc_tiling_on_sc=False),
    out_type=[
        jax.ShapeDtypeStruct((2, NP), jnp.float32),
        jax.ShapeDtypeStruct((2, NP), jnp.float32),
    ],
    scratch_types=[
        pltpu.VMEM((NP,), jnp.float32),
        pltpu.VMEM((NP,), jnp.float32),
        pltpu.VMEM((NP,), jnp.float32),
        pltpu.VMEM((NCHUNK, C), jnp.int32),
        pltpu.VMEM((NCHUNK, C), jnp.int32),
        pltpu.VMEM((C,), jnp.float32),
        pltpu.VMEM((C,), jnp.float32),
        pltpu.VMEM_SHARED((NP,), jnp.float32),
        pltpu.VMEM_SHARED((NP,), jnp.float32),
    ],
)
def _sc_b(h2_hbm, hs2_hbm, hd2_hbm, src_hbm, dst_hbm, z640_hbm,
          acc2_out, den2_out,
          h2_v, hs2_v, hd2_v, src_v, dst_v, ubA, ubB, acc2_sh, den2_sh):
    cid = lax.axis_index("c")
    sid = lax.axis_index("s")
    rowbase = (cid * 16 + sid) * NCHUNK

    pltpu.sync_copy(h2_hbm, h2_v)
    pltpu.sync_copy(hs2_hbm, hs2_v)
    pltpu.sync_copy(hd2_hbm, hd2_v)
    pltpu.sync_copy(src_hbm.at[pl.ds(rowbase, NCHUNK)], src_v)
    pltpu.sync_copy(dst_hbm.at[pl.ds(rowbase, NCHUNK)], dst_v)

    pltpu.sync_copy(z640_hbm, acc2_sh.at[pl.ds(sid * 640, 640)])
    pltpu.sync_copy(z640_hbm, den2_sh.at[pl.ds(sid * 640, 640)])
    plsc.subcore_barrier()

    def chunk_body(ci, carry):
        for g in range(C // 16):
            sv = src_v[ci, pl.ds(g * 16, 16)]
            dv = dst_v[ci, pl.ds(g * 16, 16)]
            al = (plsc.load_gather(hs2_v, [sv])
                  + plsc.load_gather(hd2_v, [dv]))
            al = jnp.where(al >= 0, al, al * jnp.float32(0.2))
            ex = jnp.exp(al)
            ubB[pl.ds(g * 16, 16)] = ex
            ubA[pl.ds(g * 16, 16)] = ex * plsc.load_gather(h2_v, [sv])
        pltpu.sync_copy(ubA, acc2_sh.at[dst_v.at[ci]], add=True)
        pltpu.sync_copy(ubB, den2_sh.at[dst_v.at[ci]], add=True)
        return carry

    lax.fori_loop(0, NCHUNK, chunk_body, 0)
    plsc.subcore_barrier()

    pltpu.sync_copy(acc2_sh.at[pl.ds(sid * 640, 640)],
                    acc2_out.at[cid, pl.ds(sid * 640, 640)])
    pltpu.sync_copy(den2_sh.at[pl.ds(sid * 640, 640)],
                    den2_out.at[cid, pl.ds(sid * 640, 640)])


def _tcc_body(acc2_ref, den2_ref, h2_ref, hs2_ref, hd2_ref, b2_ref,
              scale_ref, out_ref):
    a = acc2_ref[...]
    d = den2_ref[...]
    al = hs2_ref[...] + hd2_ref[...]
    al = jnp.where(al >= 0, al, al * jnp.float32(0.2))
    exs = jnp.exp(al)
    accf = a[0:1] + a[1:2] + exs * h2_ref[...]
    denf = d[0:1] + d[1:2] + exs
    out2 = jnp.maximum(accf / (denf + 1e-16) + b2_ref[...], 0.0)
    out_ref[...] = out2 * scale_ref[...]


def _tc_c(acc2, den2, h2r, hs2r, hd2r, b2, scale):
    return pl.pallas_call(
        _tcc_body,
        out_shape=jax.ShapeDtypeStruct((1, NP), jnp.float32),
    )(acc2, den2, h2r, hs2r, hd2r, b2, scale)


def kernel(x, edge_index, batch, W1, att_src1, att_dst1, bias1,
           W2, att_src2, att_dst2, bias2):
    f32 = jnp.float32
    x_pad = jnp.pad(x, ((0, NP - N), (0, 0)))
    w1t = jnp.zeros((128, F), f32).at[:, :41].set(W1.T)
    a2mat = (jnp.zeros((F, 2), f32)
             .at[:41, 0].set(att_src1)
             .at[:41, 1].set(att_dst1))
    hext, ab = _tc_a(x_pad, w1t, a2mat)

    pad = EP - E
    srcp = jnp.concatenate([edge_index[0], jnp.zeros((pad,), jnp.int32)])
    dstp = jnp.concatenate([edge_index[1],
                            jnp.full((pad,), NP - 1, jnp.int32)])
    src2d = srcp.reshape(EP // C, C)
    dst2d = dstp.reshape(EP // C, C)
    zrows = jnp.zeros((128, F), f32)
    acc = _sc_a(ab[:, 0], ab[:, 1], src2d, dst2d, hext, zrows)

    b1p = jnp.zeros((1, F), f32).at[0, :41].set(bias1)
    w2p = jnp.zeros((1, F), f32).at[0, :41].set(W2[0])
    a2v = jnp.stack([att_src2[0], att_dst2[0]]).reshape(1, 2)
    h2, hs2, hd2 = _tc_b(acc, ab, hext, b1p, w2p, a2v)

    h2f = h2.reshape(NP)
    hs2f = hs2.reshape(NP)
    hd2f = hd2.reshape(NP)
    z640 = jnp.zeros((640,), f32)
    acc2, den2 = _sc_b(h2f, hs2f, hd2f, src2d, dst2d, z640)

    scale = ((batch[-1] + 1) // 10).astype(f32).reshape(1, 1)
    b2 = bias2.reshape(1, 1)
    out2 = _tc_c(acc2, den2, h2f.reshape(1, NP), hs2f.reshape(1, NP),
                 hd2f.reshape(1, NP), b2, scale)
    return out2[0, :N].reshape(10, 1000)


# speedup vs baseline: 92.1107x; 1.9780x over previous
import functools

import jax
import jax.numpy as jnp
from jax import lax
from jax.experimental import pallas as pl
from jax.experimental.pallas import tpu as pltpu
from jax.experimental.pallas import tpu_sc as plsc

N = 10000
NP = 10240
E = 320000
EP = 327680
C = 80
NCHUNK = EP // 32 // C
F = 48
ONES_COL = 41

_mesh = plsc.VectorSubcoreMesh(core_axis_name="c", subcore_axis_name="s")

_GDN = lax.GatherDimensionNumbers(
    offset_dims=(), collapsed_slice_dims=(0,), start_index_map=(0,))


def _splat(vec16, idx16):
    return lax.gather(vec16, idx16[:, None], _GDN, (1,),
                      mode=lax.GatherScatterMode.PROMISE_IN_BOUNDS)


def _tca_body(x_ref, w_ref, a2_ref, h_ref, ab_ref):
    h = jnp.dot(x_ref[...], w_ref[...], preferred_element_type=jnp.float32)
    col = lax.broadcasted_iota(jnp.int32, h.shape, 1)
    hext = jnp.where(col == ONES_COL, 1.0, h)
    h_ref[...] = hext
    ab_ref[...] = jnp.dot(hext, a2_ref[...], preferred_element_type=jnp.float32)


def _tc_a(x_pad, w1t, a2mat):
    return pl.pallas_call(
        _tca_body,
        out_shape=[
            jax.ShapeDtypeStruct((NP, F), jnp.float32),
            jax.ShapeDtypeStruct((NP, 2), jnp.float32),
        ],
    )(x_pad, w1t, a2mat)


@functools.partial(
    pl.kernel,
    mesh=_mesh,
    compiler_params=pltpu.CompilerParams(needs_layout_passes=False, use_tc_tiling_on_sc=False),
    out_type=jax.ShapeDtypeStruct((2, NP, F), jnp.float32),
    scratch_types=[
        pltpu.VMEM((NP,), jnp.float32),
        pltpu.VMEM((NP,), jnp.float32),
        pltpu.VMEM((NCHUNK, C), jnp.int32),
        pltpu.VMEM((NCHUNK, C), jnp.int32),
        pltpu.VMEM((C,), jnp.float32),
        pltpu.VMEM((C, F), jnp.float32),
        pltpu.VMEM((C, F), jnp.float32),
        pltpu.VMEM_SHARED((NP, F), jnp.float32),
        pltpu.VMEM_SHARED((NP, F), jnp.float32),
        pltpu.SemaphoreType.DMA,
        pltpu.SemaphoreType.DMA,
        pltpu.SemaphoreType.DMA,
        pltpu.SemaphoreType.DMA,
    ],
)
def _sc_a(as_hbm, ad_hbm, src_hbm, dst_hbm, hext_hbm, zrows_hbm, acc_out,
          as_v, ad_v, src_v, dst_v, exbuf, rows0, rows1, hx_sh, acc_sh,
          gsem0, gsem1, ssem0, ssem1):
    cid = lax.axis_index("c")
    sid = lax.axis_index("s")
    rowbase = (cid * 16 + sid) * NCHUNK

    pltpu.sync_copy(as_hbm, as_v)
    pltpu.sync_copy(ad_hbm, ad_v)
    pltpu.sync_copy(src_hbm.at[pl.ds(rowbase, NCHUNK)], src_v)
    pltpu.sync_copy(dst_hbm.at[pl.ds(rowbase, NCHUNK)], dst_v)

    pltpu.sync_copy(hext_hbm.at[pl.ds(sid * 640, 640)],
                    hx_sh.at[pl.ds(sid * 640, 640)])
    for k in range(5):
        pltpu.sync_copy(zrows_hbm, acc_sh.at[pl.ds(sid * 640 + k * 128, 128)])
    plsc.subcore_barrier()

    lane_ids = [jnp.full((16,), l, jnp.int32) for l in range(16)]

    def compute_ex(ci):
        for g in range(C // 16):
            sv = src_v[ci, pl.ds(g * 16, 16)]
            dv = dst_v[ci, pl.ds(g * 16, 16)]
            al = (plsc.load_gather(as_v, [sv])
                  + plsc.load_gather(ad_v, [dv]))
            al = jnp.where(al >= 0, al, al * jnp.float32(0.2))
            exbuf[pl.ds(g * 16, 16)] = jnp.exp(al)

    def scale_rows(rows_v):
        for g in range(C // 16):
            exv = exbuf[pl.ds(g * 16, 16)]
            for l in range(16):
                spl = _splat(exv, lane_ids[l])
                e = g * 16 + l
                for j in range(F // 16):
                    r = rows_v[e, pl.ds(j * 16, 16)]
                    rows_v[e, pl.ds(j * 16, 16)] = r * spl

    def start_gather(ci, rows_v, sem):
        pltpu.async_copy(hx_sh.at[src_v.at[ci]], rows_v, sem)

    last = jnp.int32(NCHUNK - 1)
    start_gather(jnp.int32(0), rows0, gsem0)
    start_gather(jnp.int32(1), rows1, gsem1)

    def pair_body(i, carry):
        a = i * 2
        b = a + 1
        for (ci, rows_v, gsem, ssem) in ((a, rows0, gsem0, ssem0),
                                         (b, rows1, gsem1, ssem1)):
            compute_ex(ci)
            pltpu.make_async_copy(hx_sh.at[src_v.at[ci]], rows_v, gsem).wait()
            scale_rows(rows_v)
            pltpu.async_copy(rows_v, acc_sh.at[dst_v.at[ci]], ssem, add=True)
        for (nxt, rows_v, gsem, ssem) in ((a + 2, rows0, gsem0, ssem0),
                                          (a + 3, rows1, gsem1, ssem1)):
            pltpu.make_async_copy(rows_v, acc_sh.at[dst_v.at[jnp.int32(0)]],
                                  ssem).wait()
            start_gather(jnp.minimum(nxt, last), rows_v, gsem)
        return carry

    lax.fori_loop(0, NCHUNK // 2, pair_body, 0)
    pltpu.make_async_copy(hx_sh.at[src_v.at[jnp.int32(0)]], rows0, gsem0).wait()
    pltpu.make_async_copy(hx_sh.at[src_v.at[jnp.int32(0)]], rows1, gsem1).wait()
    plsc.subcore_barrier()

    pltpu.sync_copy(acc_sh.at[pl.ds(sid * 640, 640)],
                    acc_out.at[cid, pl.ds(sid * 640, 640)])


def _tcb_body(acc_ref, ab_ref, h_ref, b1_ref, w2_ref, a2v_ref,
              h2_ref, hs2_ref, hd2_ref):
    A = acc_ref[...]
    ab = ab_ref[...]
    al = ab[:, 0:1] + ab[:, 1:2]
    al = jnp.where(al >= 0, al, al * jnp.float32(0.2))
    exs = jnp.exp(al)
    accf = A[0] + A[1] + exs * h_ref[...]
    den = accf[:, ONES_COL:ONES_COL + 1]
    out1 = jnp.maximum(accf / (den + 1e-16) + b1_ref[...], 0.0)
    h2 = jnp.sum(out1 * w2_ref[...], axis=1, keepdims=True)
    h2_ref[...] = h2
    hs2_ref[...] = h2 * a2v_ref[0, 0]
    hd2_ref[...] = h2 * a2v_ref[0, 1]


def _tc_b(acc, ab, hext, b1p, w2p, a2v):
    return pl.pallas_call(
        _tcb_body,
        out_shape=[
            jax.ShapeDtypeStruct((NP, 1), jnp.float32),
            jax.ShapeDtypeStruct((NP, 1), jnp.float32),
            jax.ShapeDtypeStruct((NP, 1), jnp.float32),
        ],
    )(acc, ab, hext, b1p, w2p, a2v)


@functools.partial(
    pl.kernel,
    mesh=_mesh,
    compiler_params=pltpu.CompilerParams(needs_layout_passes=False, use_tc_tiling_on_sc=False),
    out_type=[
        jax.ShapeDtypeStruct((2, NP), jnp.float32),
        jax.ShapeDtypeStruct((2, NP), jnp.float32),
    ],
    scratch_types=[
        pltpu.VMEM((NP,), jnp.float32),
        pltpu.VMEM((NP,), jnp.float32),
        pltpu.VMEM((NP,), jnp.float32),
        pltpu.VMEM((NCHUNK, C), jnp.int32),
        pltpu.VMEM((NCHUNK, C), jnp.int32),
        pltpu.VMEM((C,), jnp.float32),
        pltpu.VMEM((C,), jnp.float32),
        pltpu.VMEM_SHARED((NP,), jnp.float32),
        pltpu.VMEM_SHARED((NP,), jnp.float32),
    ],
)
def _sc_b(h2_hbm, hs2_hbm, hd2_hbm, src_hbm, dst_hbm, z640_hbm,
          acc2_out, den2_out,
          h2_v, hs2_v, hd2_v, src_v, dst_v, ubA, ubB, acc2_sh, den2_sh):
    cid = lax.axis_index("c")
    sid = lax.axis_index("s")
    rowbase = (cid * 16 + sid) * NCHUNK

    pltpu.sync_copy(h2_hbm, h2_v)
    pltpu.sync_copy(hs2_hbm, hs2_v)
    pltpu.sync_copy(hd2_hbm, hd2_v)
    pltpu.sync_copy(src_hbm.at[pl.ds(rowbase, NCHUNK)], src_v)
    pltpu.sync_copy(dst_hbm.at[pl.ds(rowbase, NCHUNK)], dst_v)

    pltpu.sync_copy(z640_hbm, acc2_sh.at[pl.ds(sid * 640, 640)])
    pltpu.sync_copy(z640_hbm, den2_sh.at[pl.ds(sid * 640, 640)])
    plsc.subcore_barrier()

    def chunk_body(ci, carry):
        for g in range(C // 16):
            sv = src_v[ci, pl.ds(g * 16, 16)]
            dv = dst_v[ci, pl.ds(g * 16, 16)]
            al = (plsc.load_gather(hs2_v, [sv])
                  + plsc.load_gather(hd2_v, [dv]))
            al = jnp.where(al >= 0, al, al * jnp.float32(0.2))
            ex = jnp.exp(al)
            ubB[pl.ds(g * 16, 16)] = ex
            ubA[pl.ds(g * 16, 16)] = ex * plsc.load_gather(h2_v, [sv])
        pltpu.sync_copy(ubA, acc2_sh.at[dst_v.at[ci]], add=True)
        pltpu.sync_copy(ubB, den2_sh.at[dst_v.at[ci]], add=True)
        return carry

    lax.fori_loop(0, NCHUNK, chunk_body, 0)
    plsc.subcore_barrier()

    pltpu.sync_copy(acc2_sh.at[pl.ds(sid * 640, 640)],
                    acc2_out.at[cid, pl.ds(sid * 640, 640)])
    pltpu.sync_copy(den2_sh.at[pl.ds(sid * 640, 640)],
                    den2_out.at[cid, pl.ds(sid * 640, 640)])


def _tcc_body(acc2_ref, den2_ref, h2_ref, hs2_ref, hd2_ref, b2_ref,
              scale_ref, out_ref):
    a = acc2_ref[...]
    d = den2_ref[...]
    al = hs2_ref[...] + hd2_ref[...]
    al = jnp.where(al >= 0, al, al * jnp.float32(0.2))
    exs = jnp.exp(al)
    accf = a[0:1] + a[1:2] + exs * h2_ref[...]
    denf = d[0:1] + d[1:2] + exs
    out2 = jnp.maximum(accf / (denf + 1e-16) + b2_ref[...], 0.0)
    out_ref[...] = out2 * scale_ref[...]


def _tc_c(acc2, den2, h2r, hs2r, hd2r, b2, scale):
    return pl.pallas_call(
        _tcc_body,
        out_shape=jax.ShapeDtypeStruct((1, NP), jnp.float32),
    )(acc2, den2, h2r, hs2r, hd2r, b2, scale)


def kernel(x, edge_index, batch, W1, att_src1, att_dst1, bias1,
           W2, att_src2, att_dst2, bias2):
    f32 = jnp.float32
    x_pad = jnp.pad(x, ((0, NP - N), (0, 0)))
    w1t = jnp.zeros((128, F), f32).at[:, :41].set(W1.T)
    a2mat = (jnp.zeros((F, 2), f32)
             .at[:41, 0].set(att_src1)
             .at[:41, 1].set(att_dst1))
    hext, ab = _tc_a(x_pad, w1t, a2mat)

    pad = EP - E
    srcp = jnp.concatenate([edge_index[0], jnp.zeros((pad,), jnp.int32)])
    dstp = jnp.concatenate([edge_index[1],
                            jnp.full((pad,), NP - 1, jnp.int32)])
    src2d = srcp.reshape(EP // C, C)
    dst2d = dstp.reshape(EP // C, C)
    zrows = jnp.zeros((128, F), f32)
    acc = _sc_a(ab[:, 0], ab[:, 1], src2d, dst2d, hext, zrows)

    b1p = jnp.zeros((1, F), f32).at[0, :41].set(bias1)
    w2p = jnp.zeros((1, F), f32).at[0, :41].set(W2[0])
    a2v = jnp.stack([att_src2[0], att_dst2[0]]).reshape(1, 2)
    h2, hs2, hd2 = _tc_b(acc, ab, hext, b1p, w2p, a2v)

    h2f = h2.reshape(NP)
    hs2f = hs2.reshape(NP)
    hd2f = hd2.reshape(NP)
    z640 = jnp.zeros((640,), f32)
    acc2, den2 = _sc_b(h2f, hs2f, hd2f, src2d, dst2d, z640)

    scale = ((batch[-1] + 1) // 10).astype(f32).reshape(1, 1)
    b2 = bias2.reshape(1, 1)
    out2 = _tc_c(acc2, den2, h2f.reshape(1, NP), hs2f.reshape(1, NP),
                 hd2f.reshape(1, NP), b2, scale)
    return out2[0, :N].reshape(10, 1000)
